# Initial kernel scaffold; baseline (speedup 1.0000x reference)
import jax
import jax.numpy as jnp
from jax.experimental import pallas as pl


def kernel(input_w, adj, pad_adj_full_list, emb, W_ih_f, W_hh_f, b_ih_f, b_hh_f, W_ih_b, W_hh_b, b_ih_b, b_hh_b, basis, comp, root, rgcn_bias):
    raise NotImplementedError("write your pallas kernel here")


# speedup vs baseline: 3.1476x; 3.1476x over previous
import functools

import jax
import jax.numpy as jnp
from jax import lax
from jax.experimental import pallas as pl
from jax.experimental.pallas import tpu as pltpu
from jax.experimental.pallas import tpu_sc as plsc

B, N, T = 16, 50, 40
VOCAB, EMB, HID = 30000, 256, 512
H2 = HID // 2
NREL, NBASES = 9, 4
NN = B * N
NTOK = NN * T


_GCH = 40


def _sc_gather(emb, idx):
    info = plsc.get_sparse_core_info()
    nw = info.num_cores * info.num_subcores
    per_w = NTOK // nw
    n_ch = per_w // _GCH
    mesh = plsc.VectorSubcoreMesh(core_axis_name="c", subcore_axis_name="s")

    @functools.partial(
        pl.kernel,
        out_type=jax.ShapeDtypeStruct((NTOK, EMB), jnp.float32),
        mesh=mesh,
        scratch_types=[
            pltpu.VMEM((_GCH,), jnp.int32),
            pltpu.VMEM((_GCH, EMB), jnp.float32),
            pltpu.SemaphoreType.DMA,
        ],
    )
    def gk(table_hbm, idx_hbm, out_hbm, idx_v, rows_v, sem):
        wid = lax.axis_index("s") * info.num_cores + lax.axis_index("c")
        base = wid * per_w
        for c in range(n_ch):
            off = base + c * _GCH
            pltpu.sync_copy(idx_hbm.at[pl.ds(off, _GCH)], idx_v)
            pltpu.async_copy(table_hbm.at[idx_v], rows_v, sem).wait()
            pltpu.sync_copy(rows_v, out_hbm.at[pl.ds(off, _GCH)])

    return gk(emb, idx)


_RB = 200


def _bilstm_body(x_ref, wf_ref, wb_ref, bf_ref, bb_ref, out_ref):
    wfi = wf_ref[:EMB, :]
    wfh = wf_ref[EMB:, :]
    wbi = wb_ref[:EMB, :]
    wbh = wb_ref[EMB:, :]
    bfv = bf_ref[...]
    bbv = bb_ref[...]

    def gates(x, h, wi, wh, bv, c):
        g = (
            jnp.dot(x, wi, preferred_element_type=jnp.float32)
            + jnp.dot(h, wh, preferred_element_type=jnp.float32)
            + bv
        )
        i_ = jax.nn.sigmoid(g[:, :H2])
        f_ = jax.nn.sigmoid(g[:, H2 : 2 * H2])
        g_ = jnp.tanh(g[:, 2 * H2 : 3 * H2])
        o_ = jax.nn.sigmoid(g[:, 3 * H2 :])
        c2 = f_ * c + i_ * g_
        h2 = o_ * jnp.tanh(c2)
        return h2, c2

    def step(t, carry):
        hf, cf, hb, cb, mf, mb = carry
        xf = x_ref[pl.ds(t, 1)][0]
        xb = x_ref[pl.ds(T - 1 - t, 1)][0]
        hf, cf = gates(xf, hf, wfi, wfh, bfv, cf)
        hb, cb = gates(xb, hb, wbi, wbh, bbv, cb)
        return hf, cf, hb, cb, jnp.maximum(mf, hf), jnp.maximum(mb, hb)

    z = jnp.zeros((_RB, H2), jnp.float32)
    _, _, _, _, mf, mb = lax.fori_loop(0, T, step, (z, z, z, z, z, z))
    out_ref[:, :H2] = mf
    out_ref[:, H2:] = mb


def _bilstm(x_t, wf, wb, bf, bb):
    return pl.pallas_call(
        _bilstm_body,
        grid=(NN // _RB,),
        in_specs=[
            pl.BlockSpec((T, _RB, EMB), lambda r: (0, r, 0)),
            pl.BlockSpec((EMB + H2, 4 * H2), lambda r: (0, 0)),
            pl.BlockSpec((EMB + H2, 4 * H2), lambda r: (0, 0)),
            pl.BlockSpec((1, 4 * H2), lambda r: (0, 0)),
            pl.BlockSpec((1, 4 * H2), lambda r: (0, 0)),
        ],
        out_specs=pl.BlockSpec((_RB, HID), lambda r: (r, 0)),
        out_shape=jax.ShapeDtypeStruct((NN, HID), jnp.float32),
    )(x_t, wf, wb, bf, bb)


def _wcat_body(comp_ref, basis_ref, out_ref):
    out_ref[...] = jnp.dot(
        comp_ref[...], basis_ref[...], preferred_element_type=jnp.float32
    )


def _wcat(comp, basis):
    w = pl.pallas_call(
        _wcat_body,
        out_shape=jax.ShapeDtypeStruct((NREL, HID * HID), jnp.float32),
    )(comp, basis.reshape(NBASES, HID * HID))
    return w.reshape(NREL * HID, HID)


def _rgcn_body(node_ref, pad_ref, wcat_ref, root_ref, bias_ref, out_ref):
    node = node_ref[0]
    pad = pad_ref[0] > 0.5
    ii = lax.broadcasted_iota(jnp.int32, (N, N), 0)
    jj = lax.broadcasted_iota(jnp.int32, (N, N), 1)
    rid = (ii % 2) * 4 + (jj % 2) * 2 + (ii < jj).astype(jnp.int32)
    eye = ii == jj
    means = []
    for r in range(NREL):
        if r == NREL - 1:
            m = jnp.where((~pad) & eye, 1.0, 0.0)
        else:
            m = jnp.where(pad & (rid == r), 1.0, 0.0)
        inv = 1.0 / jnp.maximum(jnp.sum(m, axis=0), 1.0)
        ms = m * inv[None, :]
        means.append(
            lax.dot_general(
                ms, node, (((0,), (0,)), ((), ())),
                preferred_element_type=jnp.float32,
            )
        )
    meancat = jnp.concatenate(means, axis=1)
    out_ref[0] = (
        jnp.dot(node, root_ref[...], preferred_element_type=jnp.float32)
        + jnp.dot(meancat, wcat_ref[...], preferred_element_type=jnp.float32)
        + bias_ref[...]
    )


def _rgcn(node, padf, wcat, root, bias):
    return pl.pallas_call(
        _rgcn_body,
        grid=(B,),
        in_specs=[
            pl.BlockSpec((1, N, HID), lambda b: (b, 0, 0)),
            pl.BlockSpec((1, N, N), lambda b: (b, 0, 0)),
            pl.BlockSpec((NREL * HID, HID), lambda b: (0, 0)),
            pl.BlockSpec((HID, HID), lambda b: (0, 0)),
            pl.BlockSpec((1, HID), lambda b: (0, 0)),
        ],
        out_specs=pl.BlockSpec((1, N, HID), lambda b: (b, 0, 0)),
        out_shape=jax.ShapeDtypeStruct((B, N, HID), jnp.float32),
    )(node, padf, wcat, root, bias)


def kernel(
    input_w,
    adj,
    pad_adj_full_list,
    emb,
    W_ih_f,
    W_hh_f,
    b_ih_f,
    b_hh_f,
    W_ih_b,
    W_hh_b,
    b_ih_b,
    b_hh_b,
    basis,
    comp,
    root,
    rgcn_bias,
):
    del adj
    idx = input_w.reshape(NN, T).astype(jnp.int32).T.reshape(NTOK)
    x_t = _sc_gather(emb, idx).reshape(T, NN, EMB)

    wf = jnp.concatenate([W_ih_f.T, W_hh_f.T], axis=0)
    wb = jnp.concatenate([W_ih_b.T, W_hh_b.T], axis=0)
    bf = (b_ih_f + b_hh_f).reshape(1, 4 * H2)
    bb = (b_ih_b + b_hh_b).reshape(1, 4 * H2)
    node = _bilstm(x_t, wf, wb, bf, bb)

    wcat = _wcat(comp, basis)
    padf = pad_adj_full_list.astype(jnp.float32)
    out = _rgcn(
        node.reshape(B, N, HID), padf, wcat, root, rgcn_bias.reshape(1, HID)
    )
    return out
